# baseline (speedup 1.0000x reference)
import functools

import jax
import jax.numpy as jnp
from jax import lax
from jax.experimental import pallas as pl
from jax.experimental.pallas import tpu as pltpu
from jax.experimental.pallas import tpu_sc as plsc

_CPAD = 16
_NWORKERS = 32
_NBUF = 8


def _project_body(emb_ref, w_ref, idx_ref, p_ref, ipad_ref):
    ncls, d = w_ref.shape
    w_full = jnp.concatenate(
        [w_ref[...], jnp.zeros((_CPAD - ncls, d), jnp.float32)], axis=0)
    for q in range(8):
        prod = lax.dot_general(
            emb_ref[:, q, :], w_full,
            (((1,), (1,)), ((), ())),
            preferred_element_type=jnp.float32,
        )
        p_ref[:, q * _CPAD:(q + 1) * _CPAD] = prod
    rb, s_dim = idx_ref.shape
    ipad_ref[...] = jnp.concatenate(
        [idx_ref[...], jnp.zeros((rb, 256 - s_dim), jnp.int32)], axis=1)


def _project(emb3, w, inputs):
    v8, _, d = emb3.shape
    ncls = w.shape[0]
    batch, s_dim = inputs.shape
    br = 1568
    grid = pl.cdiv(v8, br)
    ib = batch // grid
    return pl.pallas_call(
        _project_body,
        grid=(grid,),
        in_specs=[
            pl.BlockSpec((br, 8, d), lambda i: (i, 0, 0)),
            pl.BlockSpec((ncls, d), lambda i: (0, 0)),
            pl.BlockSpec((ib, s_dim), lambda i: (i, 0)),
        ],
        out_specs=[
            pl.BlockSpec((br, 8 * _CPAD), lambda i: (i, 0)),
            pl.BlockSpec((ib, 256), lambda i: (i, 0)),
        ],
        out_shape=[
            jax.ShapeDtypeStruct((v8, 8 * _CPAD), jnp.float32),
            jax.ShapeDtypeStruct((batch, 256), jnp.int32),
        ],
    )(emb3, w, inputs)


def _gather_sum(p, idxp, batch, s_dim):
    k_dim = batch // _NWORKERS
    mesh = plsc.VectorSubcoreMesh(core_axis_name="c", subcore_axis_name="s")

    @functools.partial(
        pl.kernel,
        out_type=jax.ShapeDtypeStruct((_NWORKERS, s_dim, _CPAD), jnp.float32),
        mesh=mesh,
        scratch_types=[
            pltpu.VMEM((k_dim, 256), jnp.int32),
            pltpu.VMEM((s_dim, k_dim), jnp.int32),
            pltpu.VMEM((_NBUF, k_dim, _CPAD), jnp.float32),
            pltpu.VMEM((s_dim, _CPAD), jnp.float32),
            pltpu.SemaphoreType.DMA((_NBUF,)),
        ],
        compiler_params=pltpu.CompilerParams(
            use_tc_tiling_on_sc=False, needs_layout_passes=False),
    )
    def k(p_hbm, idx_hbm, out_hbm, raw_v, idx_v, rows_v, acc_v, gsems):
        cid = lax.axis_index("c")
        sid = lax.axis_index("s")
        w = cid * 16 + sid

        pltpu.sync_copy(idx_hbm.at[pl.ds(w * k_dim, k_dim)], raw_v)
        lanes = lax.iota(jnp.int32, 16)

        def transpose_pos(s):
            col = jnp.full((16,), s, jnp.int32)

            def tbody(g, carry):
                vals = plsc.load_gather(raw_v, [g * 16 + lanes, col])
                idx_v[s, pl.ds(g * 16, 16)] = vals
                return carry

            lax.fori_loop(0, k_dim // 16, tbody, 0)

        for b in range(_NBUF):
            transpose_pos(b)
            pltpu.async_copy(p_hbm.at[idx_v.at[b]], rows_v.at[b], gsems.at[b])

        def outer(g, carry):
            for b in range(_NBUF):
                s = g * _NBUF + b
                nxt = s + _NBUF
                rows = rows_v.at[b]

                @pl.when(nxt < s_dim)
                def _():
                    transpose_pos(nxt)

                pltpu.make_async_copy(
                    p_hbm.at[idx_v.at[s]], rows, gsems.at[b]).wait()

                def red(i, accs):
                    a0, a1, a2, a3 = accs
                    i4 = i * 4
                    return (a0 + rows[i4], a1 + rows[i4 + 1],
                            a2 + rows[i4 + 2], a3 + rows[i4 + 3])

                z = jnp.zeros((_CPAD,), jnp.float32)
                a0, a1, a2, a3 = lax.fori_loop(0, k_dim // 4, red, (z, z, z, z))
                acc_v[s] = (a0 + a1) + (a2 + a3)

                @pl.when(nxt < s_dim)
                def _():
                    pltpu.async_copy(
                        p_hbm.at[idx_v.at[nxt]], rows, gsems.at[b])

            return carry

        lax.fori_loop(0, s_dim // _NBUF, outer, 0)
        pltpu.sync_copy(acc_v, out_hbm.at[w])

    return k(p, idxp)


def kernel(inputs, emb_table, W, b):
    batch, s_dim = inputs.shape
    ncls, d = W.shape

    vocab = emb_table.shape[0]
    emb3 = emb_table.reshape(vocab // 8, 8, d)
    p_packed, idxp = _project(emb3, W, inputs)
    p = p_packed.reshape(vocab, _CPAD)

    partials = _gather_sum(p, idxp, batch, s_dim)
    return partials.sum(axis=0)[:, :ncls] * (1.0 / batch) + b[None, :]


# speedup vs baseline: 1.0122x; 1.0122x over previous
import functools

import jax
import jax.numpy as jnp
from jax import lax
from jax.experimental import pallas as pl
from jax.experimental.pallas import tpu as pltpu
from jax.experimental.pallas import tpu_sc as plsc

_CPAD = 16
_NWORKERS = 32
_NBUF = 8


def _project_body(emb_ref, w_ref, idx_ref, p_ref, ipad_ref):
    ncls, d = w_ref.shape
    w_full = jnp.concatenate(
        [w_ref[...], jnp.zeros((_CPAD - ncls, d), jnp.float32)], axis=0)
    for q in range(8):
        prod = lax.dot_general(
            emb_ref[:, q, :], w_full,
            (((1,), (1,)), ((), ())),
            preferred_element_type=jnp.float32,
        )
        p_ref[:, q * _CPAD:(q + 1) * _CPAD] = prod
    rb, s_dim = idx_ref.shape
    ipad_ref[...] = jnp.concatenate(
        [idx_ref[...], jnp.zeros((rb, 256 - s_dim), jnp.int32)], axis=1)


def _project(emb3, w, inputs):
    v8, _, d = emb3.shape
    ncls = w.shape[0]
    batch, s_dim = inputs.shape
    br = 1568
    grid = pl.cdiv(v8, br)
    ib = batch // grid
    return pl.pallas_call(
        _project_body,
        grid=(grid,),
        in_specs=[
            pl.BlockSpec((br, 8, d), lambda i: (i, 0, 0)),
            pl.BlockSpec((ncls, d), lambda i: (0, 0)),
            pl.BlockSpec((ib, s_dim), lambda i: (i, 0)),
        ],
        out_specs=[
            pl.BlockSpec((br, 8 * _CPAD), lambda i: (i, 0)),
            pl.BlockSpec((ib, 256), lambda i: (i, 0)),
        ],
        out_shape=[
            jax.ShapeDtypeStruct((v8, 8 * _CPAD), jnp.float32),
            jax.ShapeDtypeStruct((batch, 256), jnp.int32),
        ],
    )(emb3, w, inputs)


def _gather_sum(p, idxp, batch, s_dim):
    k_dim = batch // _NWORKERS
    mesh = plsc.VectorSubcoreMesh(core_axis_name="c", subcore_axis_name="s")

    @functools.partial(
        pl.kernel,
        out_type=jax.ShapeDtypeStruct((_NWORKERS, s_dim, _CPAD), jnp.float32),
        mesh=mesh,
        scratch_types=[
            pltpu.VMEM((k_dim * 2, 128), jnp.int32),
            pltpu.VMEM((s_dim, k_dim), jnp.int32),
            pltpu.VMEM((_NBUF, k_dim, _CPAD), jnp.float32),
            pltpu.VMEM((s_dim, _CPAD), jnp.float32),
            pltpu.SemaphoreType.DMA((_NBUF,)),
        ],
        compiler_params=pltpu.CompilerParams(
            use_tc_tiling_on_sc=False, needs_layout_passes=False),
    )
    def k(p_hbm, idx_hbm, out_hbm, raw_v, idx_v, rows_v, acc_v, gsems):
        cid = lax.axis_index("c")
        sid = lax.axis_index("s")
        w = cid * 16 + sid

        pltpu.sync_copy(idx_hbm.at[pl.ds(w * (k_dim * 2), k_dim * 2)], raw_v)
        lanes_p = lax.iota(jnp.int32, 16) * 256

        def transpose_pos(s):
            def tbody(g, carry):
                f = jnp.full((16,), g * 16 * 256 + s, jnp.int32) + lanes_p
                vals = plsc.load_gather(raw_v, [f >> 7, f & 127])
                idx_v[s, pl.ds(g * 16, 16)] = vals
                return carry

            lax.fori_loop(0, k_dim // 16, tbody, 0)

        for b in range(_NBUF):
            transpose_pos(b)
            pltpu.async_copy(p_hbm.at[idx_v.at[b]], rows_v.at[b], gsems.at[b])

        def outer(g, carry):
            for b in range(_NBUF):
                s = g * _NBUF + b
                nxt = s + _NBUF
                rows = rows_v.at[b]

                @pl.when(nxt < s_dim)
                def _():
                    transpose_pos(nxt)

                pltpu.make_async_copy(
                    p_hbm.at[idx_v.at[s]], rows, gsems.at[b]).wait()

                def red(i, accs):
                    a0, a1, a2, a3 = accs
                    i4 = i * 4
                    return (a0 + rows[i4], a1 + rows[i4 + 1],
                            a2 + rows[i4 + 2], a3 + rows[i4 + 3])

                z = jnp.zeros((_CPAD,), jnp.float32)
                a0, a1, a2, a3 = lax.fori_loop(0, k_dim // 4, red, (z, z, z, z))
                acc_v[s] = (a0 + a1) + (a2 + a3)

                @pl.when(nxt < s_dim)
                def _():
                    pltpu.async_copy(
                        p_hbm.at[idx_v.at[nxt]], rows, gsems.at[b])

            return carry

        lax.fori_loop(0, s_dim // _NBUF, outer, 0)
        pltpu.sync_copy(acc_v, out_hbm.at[w])

    return k(p, idxp)


def kernel(inputs, emb_table, W, b):
    batch, s_dim = inputs.shape
    ncls, d = W.shape

    vocab = emb_table.shape[0]
    emb3 = emb_table.reshape(vocab // 8, 8, d)
    p_packed, idxp = _project(emb3, W, inputs)
    p = p_packed.reshape(vocab, _CPAD)

    partials = _gather_sum(p, idxp.reshape(batch * 2, 128), batch, s_dim)
    return partials.sum(axis=0)[:, :ncls] * (1.0 / batch) + b[None, :]


# speedup vs baseline: 1.1361x; 1.1223x over previous
import functools

import jax
import jax.numpy as jnp
from jax import lax
from jax.experimental import pallas as pl
from jax.experimental.pallas import tpu as pltpu
from jax.experimental.pallas import tpu_sc as plsc

_CPAD = 16
_NWORKERS = 32
_NBUF = 8


def _project_body(emb_ref, w_ref, p_ref):
    ncls, d = w_ref.shape
    w_full = jnp.concatenate(
        [w_ref[...], jnp.zeros((_CPAD - ncls, d), jnp.float32)], axis=0)
    for q in range(8):
        prod = lax.dot_general(
            emb_ref[:, q, :], w_full,
            (((1,), (1,)), ((), ())),
            preferred_element_type=jnp.float32,
        )
        p_ref[:, q * _CPAD:(q + 1) * _CPAD] = prod


def _project(emb3, w):
    v8, _, d = emb3.shape
    ncls = w.shape[0]
    br = 1568
    return pl.pallas_call(
        _project_body,
        grid=(pl.cdiv(v8, br),),
        in_specs=[
            pl.BlockSpec((br, 8, d), lambda i: (i, 0, 0)),
            pl.BlockSpec((ncls, d), lambda i: (0, 0)),
        ],
        out_specs=pl.BlockSpec((br, 8 * _CPAD), lambda i: (i, 0)),
        out_shape=jax.ShapeDtypeStruct((v8, 8 * _CPAD), jnp.float32),
    )(emb3, w)


def _gather_sum(p, idx2, batch, s_dim):
    k_dim = batch // _NWORKERS
    rows_blk = k_dim * s_dim // 128
    mesh = plsc.VectorSubcoreMesh(core_axis_name="c", subcore_axis_name="s")

    @functools.partial(
        pl.kernel,
        out_type=jax.ShapeDtypeStruct((_NWORKERS, s_dim, _CPAD), jnp.float32),
        mesh=mesh,
        scratch_types=[
            pltpu.VMEM((rows_blk, 128), jnp.int32),
            pltpu.VMEM((s_dim, k_dim), jnp.int32),
            pltpu.VMEM((_NBUF, k_dim, _CPAD), jnp.float32),
            pltpu.VMEM((s_dim, _CPAD), jnp.float32),
            pltpu.SemaphoreType.DMA((_NBUF,)),
        ],
        compiler_params=pltpu.CompilerParams(
            use_tc_tiling_on_sc=False, needs_layout_passes=False),
    )
    def k(p_hbm, idx_hbm, out_hbm, raw_v, idx_v, rows_v, acc_v, gsems):
        cid = lax.axis_index("c")
        sid = lax.axis_index("s")
        w = cid * 16 + sid

        pltpu.sync_copy(idx_hbm.at[pl.ds(w * rows_blk, rows_blk)], raw_v)
        lanes_s = lax.iota(jnp.int32, 16) * s_dim

        def transpose_pos(s):
            def tbody(g, carry):
                f = jnp.full((16,), g * 16 * s_dim + s, jnp.int32) + lanes_s
                vals = plsc.load_gather(raw_v, [f >> 7, f & 127])
                idx_v[s, pl.ds(g * 16, 16)] = vals
                return carry

            lax.fori_loop(0, k_dim // 16, tbody, 0)

        for b in range(_NBUF):
            transpose_pos(b)
            pltpu.async_copy(p_hbm.at[idx_v.at[b]], rows_v.at[b], gsems.at[b])

        def outer(g, carry):
            for b in range(_NBUF):
                s = g * _NBUF + b
                nxt = s + _NBUF
                rows = rows_v.at[b]

                @pl.when(nxt < s_dim)
                def _():
                    transpose_pos(nxt)

                pltpu.make_async_copy(
                    p_hbm.at[idx_v.at[s]], rows, gsems.at[b]).wait()

                def red(i, accs):
                    a0, a1, a2, a3 = accs
                    i4 = i * 4
                    return (a0 + rows[i4], a1 + rows[i4 + 1],
                            a2 + rows[i4 + 2], a3 + rows[i4 + 3])

                z = jnp.zeros((_CPAD,), jnp.float32)
                a0, a1, a2, a3 = lax.fori_loop(0, k_dim // 4, red, (z, z, z, z))
                acc_v[s] = (a0 + a1) + (a2 + a3)

                @pl.when(nxt < s_dim)
                def _():
                    pltpu.async_copy(
                        p_hbm.at[idx_v.at[nxt]], rows, gsems.at[b])

            return carry

        lax.fori_loop(0, s_dim // _NBUF, outer, 0)
        pltpu.sync_copy(acc_v, out_hbm.at[w])

    return k(p, idx2)


def kernel(inputs, emb_table, W, b):
    batch, s_dim = inputs.shape
    ncls, d = W.shape

    vocab = emb_table.shape[0]
    emb3 = emb_table.reshape(vocab // 8, 8, d)
    p = _project(emb3, W).reshape(vocab, _CPAD)

    partials = _gather_sum(p, inputs.reshape(batch * s_dim // 128, 128),
                           batch, s_dim)
    return partials.sum(axis=0)[:, :ncls] * (1.0 / batch) + b[None, :]
